# speedup vs baseline: 1.1080x; 1.1080x over previous
import functools

import jax
import jax.numpy as jnp
from jax import lax
from jax.experimental import pallas as pl
from jax.experimental.pallas import tpu as pltpu
from jax.experimental.pallas import tpu_sc as plsc

NC = 2
NS = 16
NW = NC * NS
LANES = 16


@functools.lru_cache(maxsize=None)
def _build(tok, seq, d, ch):
    tpw = tok // NW
    n_chunks = tpw // ch
    mesh = plsc.VectorSubcoreMesh(
        core_axis_name="c", subcore_axis_name="s",
        num_cores=NC, num_subcores=NS)

    @functools.partial(
        pl.kernel,
        out_type=jax.ShapeDtypeStruct((tok, d), jnp.float32),
        mesh=mesh,
        scratch_types=[
            pltpu.VMEM((ch,), jnp.int32),
            pltpu.VMEM((ch, d), jnp.float32),
            pltpu.VMEM((ch, d), jnp.float32),
            pltpu.SemaphoreType.DMA,
        ],
    )
    def emb(ids_hbm, wte_hbm, wpe_hbm, out_hbm, idx_v, tok_v, pos_v, sem):
        wid = lax.axis_index("s") * NC + lax.axis_index("c")
        base = wid * tpw
        pos_base = lax.rem(base, seq)

        for c in range(n_chunks):
            off = base + c * ch
            poff = pos_base + c * ch
            pltpu.sync_copy(ids_hbm.at[pl.ds(off, ch)], idx_v)
            gat = pltpu.async_copy(wte_hbm.at[idx_v], tok_v, sem)
            pltpu.sync_copy(wpe_hbm.at[pl.ds(poff, ch)], pos_v)
            gat.wait()

            def add_row(r, carry):
                for i in range(d // LANES):
                    sl = pl.ds(i * LANES, LANES)
                    tok_v[r, sl] = tok_v[r, sl] + pos_v[r, sl]
                return carry

            lax.fori_loop(0, ch, add_row, 0)
            pltpu.sync_copy(tok_v, out_hbm.at[pl.ds(off, ch)])

    return emb


def kernel(input_ids, wte, wpe):
    b, s = input_ids.shape
    d = wte.shape[1]
    ids = input_ids.reshape(-1)
    emb = _build(b * s, s, d, 32)
    out = emb(ids, wte, wpe)
    return out.reshape(b, s, d)


# speedup vs baseline: 1.5495x; 1.3985x over previous
import functools

import jax
import jax.numpy as jnp
from jax import lax
from jax.experimental import pallas as pl
from jax.experimental.pallas import tpu as pltpu
from jax.experimental.pallas import tpu_sc as plsc

NC = 2
NS = 16
NW = NC * NS
LANES = 16
CH = 32


@functools.lru_cache(maxsize=None)
def _build(nb, seq, d):
    pw = seq // NW
    n_h = pw // CH
    n_chunks = n_h * nb
    mesh = plsc.VectorSubcoreMesh(
        core_axis_name="c", subcore_axis_name="s",
        num_cores=NC, num_subcores=NS)

    @functools.partial(
        pl.kernel,
        out_type=jax.ShapeDtypeStruct((nb * seq, d), jnp.float32),
        mesh=mesh,
        scratch_types=[
            pltpu.VMEM((CH,), jnp.int32),
            pltpu.VMEM((CH,), jnp.int32),
            pltpu.VMEM((CH, d), jnp.float32),
            pltpu.VMEM((CH, d), jnp.float32),
            pltpu.VMEM((CH, d), jnp.float32),
            pltpu.SemaphoreType.DMA,
            pltpu.SemaphoreType.DMA,
            pltpu.SemaphoreType.DMA,
            pltpu.SemaphoreType.DMA,
        ],
    )
    def emb(ids_hbm, wte_hbm, wpe_hbm, out_hbm, idx0, idx1, tok0, tok1,
            pos_v, sg0, sg1, ss0, ss1):
        idx = [idx0, idx1]
        tok = [tok0, tok1]
        sg = [sg0, sg1]
        ss = [ss0, ss1]
        wid = lax.axis_index("s") * NC + lax.axis_index("c")
        pbase = wid * pw

        def off(k):
            h, b = k // nb, k % nb
            return b * seq + pbase + h * CH

        def add(p):
            def body(r, carry):
                for i in range(d // LANES):
                    sl = pl.ds(i * LANES, LANES)
                    tok[p][r, sl] = tok[p][r, sl] + pos_v[r, sl]
                return carry
            lax.fori_loop(0, CH, body, 0)

        g = [None, None]
        s = [None, None]
        pltpu.sync_copy(wpe_hbm.at[pl.ds(pbase, CH)], pos_v)
        pltpu.sync_copy(ids_hbm.at[pl.ds(off(0), CH)], idx[0])
        g[0] = pltpu.async_copy(wte_hbm.at[idx[0]], tok[0], sg[0])

        for k in range(1, n_chunks):
            p = k & 1
            q = 1 - p
            pltpu.sync_copy(ids_hbm.at[pl.ds(off(k), CH)], idx[p])
            if s[p] is not None:
                s[p].wait()
            g[p] = pltpu.async_copy(wte_hbm.at[idx[p]], tok[p], sg[p])
            g[q].wait()
            add(q)
            s[q] = pltpu.async_copy(tok[q], out_hbm.at[pl.ds(off(k - 1), CH)],
                                    ss[q])
            if k % nb == 0:
                pltpu.sync_copy(wpe_hbm.at[pl.ds(pbase + (k // nb) * CH, CH)],
                                pos_v)

        p = (n_chunks - 1) & 1
        g[p].wait()
        add(p)
        s[p] = pltpu.async_copy(tok[p], out_hbm.at[pl.ds(off(n_chunks - 1), CH)],
                                ss[p])
        s[1 - p].wait()
        s[p].wait()

    return emb


def kernel(input_ids, wte, wpe):
    b, s = input_ids.shape
    d = wte.shape[1]
    ids = input_ids.reshape(-1)
    emb = _build(b, s, d)
    out = emb(ids, wte, wpe)
    return out.reshape(b, s, d)
